# Initial kernel scaffold; baseline (speedup 1.0000x reference)
import jax
import jax.numpy as jnp
from jax.experimental import pallas as pl


def kernel(x, edge_index, W1, b1, W2, b2):
    raise NotImplementedError("write your pallas kernel here")


# speedup vs baseline: 12.1054x; 12.1054x over previous
import functools

import jax
import jax.numpy as jnp
from jax import lax
from jax.experimental import pallas as pl
from jax.experimental.pallas import tpu as pltpu
from jax.experimental.pallas import tpu_sc as plsc

N_NODES_C = 10000
D_C = 128
NC = 2
NS = 16
NW = NC * NS
CHUNK = 128
ROWS_PAD = 10240
ROWS_PER_TILE = ROWS_PAD // NS
ZROWS = 64
DEG_W = 128

_mesh = plsc.VectorSubcoreMesh(core_axis_name="c", subcore_axis_name="s")


def _deg_kernel(dst_hbm, ones_hbm, zeros_hbm, out_hbm, deg_sp, idx_v, ones_v, zb_v):
  c = lax.axis_index("c")
  s = lax.axis_index("s")
  nch = dst_hbm.shape[2]
  pltpu.sync_copy(zeros_hbm, zb_v)

  def zbody(i, carry):
    pltpu.sync_copy(zb_v, deg_sp.at[pl.ds(s * ROWS_PER_TILE + i * ZROWS, ZROWS)])
    return carry

  lax.fori_loop(0, ROWS_PER_TILE // ZROWS, zbody, 0)
  pltpu.sync_copy(ones_hbm, ones_v)
  pltpu.sync_copy(dst_hbm.at[c, s], idx_v)
  plsc.subcore_barrier()

  def body(j, carry):
    pltpu.sync_copy(ones_v, deg_sp.at[idx_v.at[j]], add=True)
    return carry

  lax.fori_loop(0, nch, body, 0)
  plsc.subcore_barrier()

  def wbody(i, carry):
    sl = pl.ds(s * ROWS_PER_TILE + i * ZROWS, ZROWS)
    pltpu.sync_copy(deg_sp.at[sl], zb_v)
    pltpu.sync_copy(zb_v, out_hbm.at[c, sl])
    return carry

  lax.fori_loop(0, ROWS_PER_TILE // ZROWS, wbody, 0)


def _agg_kernel(z_hbm, src_hbm, dst_hbm, zeros_hbm, out_hbm,
                acc_sp, src_v, dst_v, rows_v, zb_v):
  c = lax.axis_index("c")
  s = lax.axis_index("s")
  nch = src_hbm.shape[2]
  pltpu.sync_copy(zeros_hbm, zb_v)

  def zbody(i, carry):
    pltpu.sync_copy(zb_v, acc_sp.at[pl.ds(s * ROWS_PER_TILE + i * ZROWS, ZROWS)])
    return carry

  lax.fori_loop(0, ROWS_PER_TILE // ZROWS, zbody, 0)
  pltpu.sync_copy(src_hbm.at[c, s], src_v)
  pltpu.sync_copy(dst_hbm.at[c, s], dst_v)
  plsc.subcore_barrier()

  def body(j, carry):
    pltpu.sync_copy(z_hbm.at[src_v.at[j]], rows_v)
    pltpu.sync_copy(rows_v, acc_sp.at[dst_v.at[j]], add=True)
    return carry

  lax.fori_loop(0, nch, body, 0)
  plsc.subcore_barrier()

  def wbody(i, carry):
    sl = pl.ds(s * ROWS_PER_TILE + i * ZROWS, ZROWS)
    pltpu.sync_copy(acc_sp.at[sl], zb_v)
    pltpu.sync_copy(zb_v, out_hbm.at[c, sl])
    return carry

  lax.fori_loop(0, ROWS_PER_TILE // ZROWS, wbody, 0)


def _make_sc_calls(n_chunks):
  deg_call = pl.kernel(
      _deg_kernel,
      out_type=jax.ShapeDtypeStruct((NC, ROWS_PAD, DEG_W), jnp.float32),
      mesh=_mesh,
      scratch_types=[
          pltpu.VMEM_SHARED((ROWS_PAD, DEG_W), jnp.float32),
          pltpu.VMEM((n_chunks, CHUNK), jnp.int32),
          pltpu.VMEM((CHUNK, DEG_W), jnp.float32),
          pltpu.VMEM((ZROWS, DEG_W), jnp.float32),
      ],
  )
  agg_call = pl.kernel(
      _agg_kernel,
      out_type=jax.ShapeDtypeStruct((NC, ROWS_PAD, D_C), jnp.float32),
      mesh=_mesh,
      scratch_types=[
          pltpu.VMEM_SHARED((ROWS_PAD, D_C), jnp.float32),
          pltpu.VMEM((n_chunks, CHUNK), jnp.int32),
          pltpu.VMEM((n_chunks, CHUNK), jnp.int32),
          pltpu.VMEM((CHUNK, D_C), jnp.float32),
          pltpu.VMEM((ZROWS, D_C), jnp.float32),
      ],
  )
  return deg_call, agg_call


_BM = 1000


def _tc_a_body(x_ref, w_ref, d0_ref, d1_ref, z_ref, dis_ref):
  deg = d0_ref[...] + d1_ref[...] + 1.0
  dis = jax.lax.rsqrt(deg)
  dis_ref[...] = dis
  z_ref[...] = jnp.dot(x_ref[...], w_ref[...],
                       preferred_element_type=jnp.float32) * dis


def _tc_b_body(a0_ref, a1_ref, z1_ref, dis_ref, b_ref, w_ref, z2_ref):
  dis = dis_ref[...]
  h = (a0_ref[...] + a1_ref[...] + z1_ref[...]) * dis + b_ref[...]
  h = jnp.maximum(h, 0.0)
  z2_ref[...] = jnp.dot(h, w_ref[...], preferred_element_type=jnp.float32) * dis


def _tc_c_body(a0_ref, a1_ref, z2_ref, dis_ref, b_ref, out_ref):
  out_ref[...] = (a0_ref[...] + a1_ref[...] + z2_ref[...]) * dis_ref[...] + b_ref[...]


def _row_spec(width):
  return pl.BlockSpec((_BM, width), lambda i: (i, 0))


def _full_spec(rows, cols):
  return pl.BlockSpec((rows, cols), lambda i: (0, 0))


def _tc_stage_a(x, w1, d0, d1):
  n = x.shape[0]
  grid = (n // _BM,)
  return pl.pallas_call(
      _tc_a_body,
      grid=grid,
      in_specs=[_row_spec(D_C), _full_spec(D_C, D_C), _row_spec(1), _row_spec(1)],
      out_specs=[_row_spec(D_C), _row_spec(1)],
      out_shape=[jax.ShapeDtypeStruct((n, D_C), jnp.float32),
                 jax.ShapeDtypeStruct((n, 1), jnp.float32)],
  )(x, w1, d0, d1)


def _tc_stage_b(a0, a1, z1, dis, b1, w2):
  n = z1.shape[0]
  grid = (n // _BM,)
  return pl.pallas_call(
      _tc_b_body,
      grid=grid,
      in_specs=[_row_spec(D_C), _row_spec(D_C), _row_spec(D_C), _row_spec(1),
                _full_spec(1, D_C), _full_spec(D_C, D_C)],
      out_specs=_row_spec(D_C),
      out_shape=jax.ShapeDtypeStruct((n, D_C), jnp.float32),
  )(a0, a1, z1, dis, b1, w2)


def _tc_stage_c(a0, a1, z2, dis, b2):
  n = z2.shape[0]
  grid = (n // _BM,)
  return pl.pallas_call(
      _tc_c_body,
      grid=grid,
      in_specs=[_row_spec(D_C), _row_spec(D_C), _row_spec(D_C), _row_spec(1),
                _full_spec(1, D_C)],
      out_specs=_row_spec(D_C),
      out_shape=jax.ShapeDtypeStruct((n, D_C), jnp.float32),
  )(a0, a1, z2, dis, b2)


def kernel(x, edge_index, W1, b1, W2, b2):
  n = x.shape[0]
  e = edge_index.shape[1]
  n_chunks = -(-e // (NW * CHUNK))
  e_pad = NW * n_chunks * CHUNK
  src = edge_index[0].astype(jnp.int32)
  dst = edge_index[1].astype(jnp.int32)
  pad = e_pad - e
  src_p = jnp.concatenate([src, jnp.zeros((pad,), jnp.int32)]
                          ).reshape(NC, NS, n_chunks, CHUNK)
  dst_p = jnp.concatenate([dst, jnp.full((pad,), n, jnp.int32)]
                          ).reshape(NC, NS, n_chunks, CHUNK)

  ones_rows = jnp.ones((CHUNK, DEG_W), jnp.float32)
  zeros_rows = jnp.zeros((ZROWS, D_C), jnp.float32)

  deg_call, agg_call = _make_sc_calls(n_chunks)

  deg_parts = deg_call(dst_p, ones_rows, zeros_rows)
  d0 = lax.slice(deg_parts, (0, 0, 0), (1, n, 1)).reshape(n, 1)
  d1 = lax.slice(deg_parts, (1, 0, 0), (2, n, 1)).reshape(n, 1)

  z1, dis = _tc_stage_a(x, W1, d0, d1)

  acc1 = agg_call(z1, src_p, dst_p, zeros_rows)
  a0 = lax.slice(acc1, (0, 0, 0), (1, n, D_C)).reshape(n, D_C)
  a1 = lax.slice(acc1, (1, 0, 0), (2, n, D_C)).reshape(n, D_C)

  b1r = b1.reshape(1, D_C)
  z2 = _tc_stage_b(a0, a1, z1, dis, b1r, W2)

  acc2 = agg_call(z2, src_p, dst_p, zeros_rows)
  c0 = lax.slice(acc2, (0, 0, 0), (1, n, D_C)).reshape(n, D_C)
  c1 = lax.slice(acc2, (1, 0, 0), (2, n, D_C)).reshape(n, D_C)

  b2r = b2.reshape(1, D_C)
  return _tc_stage_c(c0, c1, z2, dis, b2r)
